# baseline (speedup 1.0000x reference)
import functools

import jax
import jax.numpy as jnp
from jax import lax
from jax.experimental import pallas as pl
from jax.experimental.pallas import tpu as pltpu
from jax.experimental.pallas import tpu_sc as plsc

BATCH = 16384
Z_DIM = 64
_NUM_CORES = 2
_NUM_SUBCORES = 16
_NW = _NUM_CORES * _NUM_SUBCORES
_BPW = BATCH // _NW


def _lookup_body(u_hbm, mean_hbm, logvar_hbm, out_mean, out_logvar,
                 idx_v, mean_v, logvar_v, sem):
  wid = lax.axis_index("s") * _NUM_CORES + lax.axis_index("c")
  base = wid * _BPW
  pltpu.sync_copy(u_hbm.at[pl.ds(base, _BPW)], idx_v)
  cp_m = pltpu.async_copy(mean_hbm.at[idx_v], mean_v, sem)
  cp_l = pltpu.async_copy(logvar_hbm.at[idx_v], logvar_v, sem)
  cp_m.wait()
  cp_l.wait()
  pltpu.sync_copy(mean_v, out_mean.at[pl.ds(base, _BPW)])
  pltpu.sync_copy(logvar_v, out_logvar.at[pl.ds(base, _BPW)])


@jax.jit
def kernel(u, mean_table, logvar_table):
  mesh = plsc.VectorSubcoreMesh(core_axis_name="c", subcore_axis_name="s")
  out = jax.ShapeDtypeStruct((BATCH, Z_DIM), jnp.float32)
  run = pl.kernel(
      _lookup_body,
      out_type=(out, out),
      mesh=mesh,
      scratch_types=[
          pltpu.VMEM((_BPW,), jnp.int32),
          pltpu.VMEM((_BPW, Z_DIM), jnp.float32),
          pltpu.VMEM((_BPW, Z_DIM), jnp.float32),
          pltpu.SemaphoreType.DMA,
      ],
      compiler_params=pltpu.CompilerParams(use_tc_tiling_on_sc=False),
  )
  return run(u.astype(jnp.int32), mean_table, logvar_table)
